# baseline (speedup 1.0000x reference)
import functools

import jax
import jax.numpy as jnp
from jax import lax
from jax.experimental import pallas as pl
from jax.experimental.pallas import tpu as pltpu
from jax.experimental.pallas import tpu_sc as plsc

VOCAB = 100000
EMB_DIM = 100
BATCH = 16384
SEQ = 50
SEQ_PAD = 56

NUM_WORKERS = 32
B_W = BATCH // NUM_WORKERS
IDX_W = B_W * SEQ_PAD
OUT_W = B_W * EMB_DIM
L = 16
EMB_PAD = 128
ROW_W = EMB_PAD // 2
NGRP = ROW_W // L


def _count_nonzero(idx_ref, off):
    lane = lax.iota(jnp.int32, 16)
    v0 = idx_ref[pl.ds(off, L)]
    v1 = idx_ref[pl.ds(off + 16, L)]
    v2 = idx_ref[pl.ds(off + 32, L)]
    v3 = idx_ref[pl.ds(off + 34, L)]
    one = jnp.float32(1.0)
    zero = jnp.float32(0.0)
    c = jnp.where(v0 != 0, one, zero)
    c += jnp.where(v1 != 0, one, zero)
    c += jnp.where(v2 != 0, one, zero)
    c += jnp.where((v3 != 0) & (lane >= 14), one, zero)
    for stride in (8, 4, 2, 1):
        c = c + c.at[lane ^ stride].get(mode="promise_in_bounds")
    return c


def _sc_pool(table, x_flat):
    mesh = plsc.VectorSubcoreMesh(core_axis_name="c", subcore_axis_name="s")

    @functools.partial(
        pl.kernel,
        mesh=mesh,
        out_type=jax.ShapeDtypeStruct((BATCH * EMB_DIM,), jnp.float32),
        compiler_params=pltpu.CompilerParams(use_tc_tiling_on_sc=False,
                                             needs_layout_passes=False),
        scratch_types=[
            pltpu.VMEM((IDX_W,), jnp.int32),
            pltpu.VMEM((SEQ, ROW_W), jnp.float32),
            pltpu.VMEM((SEQ, ROW_W), jnp.float32),
            pltpu.VMEM((SEQ, ROW_W), jnp.float32),
            pltpu.VMEM((OUT_W + EMB_PAD,), jnp.float32),
            pltpu.SemaphoreType.DMA,
            pltpu.SemaphoreType.DMA,
            pltpu.SemaphoreType.DMA,
        ],
    )
    def k(table_hbm, x_hbm, out_hbm, idx_v, buf0, buf1, buf2, stage_v,
          sem0, sem1, sem2):
        wid = lax.axis_index("s") * 2 + lax.axis_index("c")
        bufs = (buf0, buf1, buf2)
        sems = (sem0, sem1, sem2)
        lane = lax.iota(jnp.int32, 16)

        pltpu.sync_copy(x_hbm.at[pl.ds(wid * IDX_W, IDX_W)], idx_v)

        def start_gather(b, buf, sem):
            pltpu.make_async_copy(
                table_hbm.at[idx_v.at[pl.ds(b * SEQ_PAD, SEQ)]], buf, sem
            ).start()

        def wait_gather(buf, sem):
            pltpu.make_async_copy(
                table_hbm.at[idx_v.at[pl.ds(0, SEQ)]], buf, sem
            ).wait()

        def process(b, buf):
            cnt = _count_nonzero(idx_v, b * SEQ_PAD)
            scale = 1.0 / (float(SEQ) * jnp.maximum(cnt, 1.0))
            base = b * EMB_DIM
            for g in range(NGRP):
                off = g * L
                pe = [None, None]
                po = [None, None]
                for s in range(SEQ):
                    w = buf[s, pl.ds(off, L)]
                    e, o = plsc.unpack(
                        plsc.bitcast(w, jnp.bfloat16),
                        format=plsc.PackFormat.INTERLEAVED,
                        preferred_element_type=jnp.float32,
                    )
                    ch = s % 2
                    pe[ch] = e if pe[ch] is None else pe[ch] + e
                    po[ch] = o if po[ch] is None else po[ch] + o
                acc_e = (pe[0] + pe[1]) * scale
                acc_o = (po[0] + po[1]) * scale
                idx_e = base + 2 * g * L + 2 * lane
                idx_o = idx_e + 1
                if g == NGRP - 1:
                    m = lane < 2
                    plsc.store_scatter(stage_v, [idx_e], acc_e, mask=m)
                    plsc.store_scatter(stage_v, [idx_o], acc_o, mask=m)
                else:
                    plsc.store_scatter(stage_v, [idx_e], acc_e)
                    plsc.store_scatter(stage_v, [idx_o], acc_o)

        for p in range(3):
            start_gather(p, bufs[p], sems[p])

        def body(i, carry):
            for par in range(3):
                b = i * 3 + par
                wait_gather(bufs[par], sems[par])
                process(b, bufs[par])
                nb = jnp.minimum(b + 3, B_W - 1)
                start_gather(nb, bufs[par], sems[par])
            return carry

        lax.fori_loop(0, B_W // 3, body, 0)
        for p in range(3):
            wait_gather(bufs[p], sems[p])
            if p < 2:
                process(B_W - 2 + p, bufs[p])

        pltpu.sync_copy(stage_v.at[pl.ds(0, OUT_W)], out_hbm.at[pl.ds(wid * OUT_W, OUT_W)])

    return k(table, x_flat)


def kernel(table, x):
    table_bf = lax.concatenate(
        [table.astype(jnp.bfloat16),
         jnp.zeros((VOCAB, EMB_PAD - EMB_DIM), jnp.bfloat16)], 1)
    table_w = lax.bitcast_convert_type(
        table_bf.reshape(VOCAB, ROW_W, 2), jnp.float32)
    x_pad = jnp.pad(x.astype(jnp.int32), ((0, 0), (0, SEQ_PAD - SEQ)))
    x_flat = x_pad.reshape(-1)
    out = _sc_pool(table_w, x_flat)
    return out.reshape(BATCH, EMB_DIM)


# speedup vs baseline: 1.9136x; 1.9136x over previous
import functools

import jax
import jax.numpy as jnp
from jax import lax
from jax.experimental import pallas as pl
from jax.experimental.pallas import tpu as pltpu
from jax.experimental.pallas import tpu_sc as plsc

VOCAB = 100000
EMB_DIM = 100
BATCH = 16384
SEQ = 50
SEQ_PAD = 56

NUM_WORKERS = 32
B_W = BATCH // NUM_WORKERS
IDX_W = B_W * SEQ_PAD
OUT_W = B_W * EMB_DIM
L = 16
EMB_PAD = 128
ROW_W = EMB_PAD // 2
NGRP = ROW_W // L


def _count_nonzero(idx_ref, off):
    lane = lax.iota(jnp.int32, 16)
    v0 = idx_ref[pl.ds(off, L)]
    v1 = idx_ref[pl.ds(off + 16, L)]
    v2 = idx_ref[pl.ds(off + 32, L)]
    v3 = idx_ref[pl.ds(off + 34, L)]
    one = jnp.float32(1.0)
    zero = jnp.float32(0.0)
    c = jnp.where(v0 != 0, one, zero)
    c += jnp.where(v1 != 0, one, zero)
    c += jnp.where(v2 != 0, one, zero)
    c += jnp.where((v3 != 0) & (lane >= 14), one, zero)
    for stride in (8, 4, 2, 1):
        c = c + c.at[lane ^ stride].get(mode="promise_in_bounds")
    return c


def _sc_pool(table, x_flat):
    mesh = plsc.VectorSubcoreMesh(core_axis_name="c", subcore_axis_name="s")

    @functools.partial(
        pl.kernel,
        mesh=mesh,
        out_type=jax.ShapeDtypeStruct((BATCH * EMB_DIM,), jnp.float32),
        compiler_params=pltpu.CompilerParams(use_tc_tiling_on_sc=False,
                                             needs_layout_passes=False),
        scratch_types=[
            pltpu.VMEM((IDX_W,), jnp.int32),
            pltpu.VMEM((SEQ, ROW_W), jnp.float32),
            pltpu.VMEM((SEQ, ROW_W), jnp.float32),
            pltpu.VMEM((SEQ, ROW_W), jnp.float32),
            pltpu.VMEM((OUT_W + EMB_PAD,), jnp.float32),
            pltpu.SemaphoreType.DMA,
            pltpu.SemaphoreType.DMA,
            pltpu.SemaphoreType.DMA,
        ],
    )
    def k(table_hbm, x_hbm, out_hbm, idx_v, buf0, buf1, buf2, stage_v,
          sem0, sem1, sem2):
        wid = lax.axis_index("s") * 2 + lax.axis_index("c")
        bufs = (buf0, buf1, buf2)
        sems = (sem0, sem1, sem2)
        lane = lax.iota(jnp.int32, 16)

        pltpu.sync_copy(x_hbm.at[pl.ds(wid * IDX_W, IDX_W)], idx_v)

        def start_gather(b, buf, sem):
            pltpu.make_async_copy(
                table_hbm.at[idx_v.at[pl.ds(b * SEQ_PAD, SEQ)]], buf, sem
            ).start()

        def wait_gather(buf, sem):
            pltpu.make_async_copy(
                table_hbm.at[idx_v.at[pl.ds(0, SEQ)]], buf, sem
            ).wait()

        def process(b, buf):
            cnt = _count_nonzero(idx_v, b * SEQ_PAD)
            scale = 1.0 / (float(SEQ) * jnp.maximum(cnt, 1.0))
            base = b * EMB_DIM
            for g in range(NGRP):
                off = g * L
                lo_cols = g * L
                hi_cols = ROW_W + g * L
                hi_real = hi_cols < EMB_DIM
                pe = [None, None]
                po = [None, None]
                for s in range(SEQ):
                    w = buf[s, pl.ds(off, L)]
                    lo, hi = plsc.unpack(
                        plsc.bitcast(w, jnp.bfloat16),
                        format=plsc.PackFormat.INTERLEAVED,
                        preferred_element_type=jnp.float32,
                    )
                    ch = s % 2
                    pe[ch] = lo if pe[ch] is None else pe[ch] + lo
                    if hi_real:
                        po[ch] = hi if po[ch] is None else po[ch] + hi
                stage_v[pl.ds(base + lo_cols, L)] = (pe[0] + pe[1]) * scale
                if hi_real:
                    stage_v[pl.ds(base + hi_cols, L)] = (po[0] + po[1]) * scale

        for p in range(3):
            start_gather(p, bufs[p], sems[p])

        def body(i, carry):
            for par in range(3):
                b = i * 3 + par
                wait_gather(bufs[par], sems[par])
                process(b, bufs[par])
                nb = jnp.minimum(b + 3, B_W - 1)
                start_gather(nb, bufs[par], sems[par])
            return carry

        lax.fori_loop(0, B_W // 3, body, 0)
        for p in range(3):
            wait_gather(bufs[p], sems[p])
            if p < 2:
                process(B_W - 2 + p, bufs[p])

        pltpu.sync_copy(stage_v.at[pl.ds(0, OUT_W)], out_hbm.at[pl.ds(wid * OUT_W, OUT_W)])

    return k(table, x_flat)


def kernel(table, x):
    table = table.astype(jnp.float32)
    lo = table[:, :ROW_W]
    hi = jnp.pad(table[:, ROW_W:], ((0, 0), (0, 2 * ROW_W - EMB_DIM)))
    lo16 = lax.bitcast_convert_type(lo.astype(jnp.bfloat16), jnp.uint16)
    hi16 = lax.bitcast_convert_type(hi.astype(jnp.bfloat16), jnp.uint16)
    packed = (lo16.astype(jnp.uint32)
              | (hi16.astype(jnp.uint32) << jnp.uint32(16)))
    table_w = lax.bitcast_convert_type(packed, jnp.float32)
    x_pad = jnp.pad(x.astype(jnp.int32), ((0, 0), (0, SEQ_PAD - SEQ)))
    x_flat = x_pad.reshape(-1)
    out = _sc_pool(table_w, x_flat)
    return out.reshape(BATCH, EMB_DIM)
